# baseline (speedup 1.0000x reference)
import functools

import jax
import jax.numpy as jnp
from jax import lax
from jax.experimental import pallas as pl
from jax.experimental.pallas import tpu as pltpu
from jax.experimental.pallas import tpu_sc as plsc

_C = 16
_NBUF = 4


def _gather_rows(table, idx3, B, D):
    info = plsc.get_sparse_core_info()
    NW = info.num_cores * info.num_subcores
    b_per_w = B // NW
    C = _C
    n_chunks = b_per_w // C
    assert b_per_w % C == 0 and n_chunks % _NBUF == 0

    mesh = plsc.VectorSubcoreMesh(core_axis_name="c", subcore_axis_name="s")

    @functools.partial(
        pl.kernel,
        mesh=mesh,
        out_type=jax.ShapeDtypeStruct((B, D), jnp.float32),
        scratch_types=[
            pltpu.VMEM((n_chunks, C), jnp.int32),
            *[pltpu.VMEM((C, D), jnp.float32) for _ in range(_NBUF)],
            *[pltpu.SemaphoreType.DMA for _ in range(2 * _NBUF)],
        ],
    )
    def k(idx_hbm, table_hbm, out_hbm, idx_v, *bufs_and_sems):
        rows = bufs_and_sems[:_NBUF]
        gsem = bufs_and_sems[_NBUF : 2 * _NBUF]
        osem = bufs_and_sems[2 * _NBUF :]
        wid = lax.axis_index("s") * info.num_cores + lax.axis_index("c")
        base = pl.multiple_of(wid * b_per_w, b_per_w)

        pltpu.sync_copy(idx_hbm.at[wid], idx_v)
        pltpu.async_copy(table_hbm.at[idx_v.at[0]], rows[0], gsem[0])

        def body(g, carry):
            for b in range(_NBUF):
                j = g * _NBUF + b
                pltpu.make_async_copy(
                    table_hbm.at[idx_v.at[j]], rows[b], gsem[b]
                ).wait()
                pltpu.async_copy(
                    rows[b], out_hbm.at[pl.ds(base + j * C, C)], osem[b]
                )
                jn = jnp.minimum(j + 1, n_chunks - 1)
                nb = (b + 1) % _NBUF

                @pl.when(jnp.logical_or(g > 0, b >= _NBUF - 1))
                def _wait_prev_out():
                    pltpu.make_async_copy(
                        rows[nb], out_hbm.at[pl.ds(base, C)], osem[nb]
                    ).wait()

                pltpu.async_copy(
                    table_hbm.at[idx_v.at[jn]], rows[nb], gsem[nb]
                )
            return carry

        lax.fori_loop(0, n_chunks // _NBUF, body, 0)
        pltpu.make_async_copy(
            table_hbm.at[idx_v.at[0]], rows[0], gsem[0]
        ).wait()
        for b in range(1, _NBUF):
            pltpu.make_async_copy(
                rows[b], out_hbm.at[pl.ds(base, C)], osem[b]
            ).wait()

    return k(idx3, table)


def kernel(indices, emb_weight):
    Bo, S = indices.shape
    V, D = emb_weight.shape
    B = Bo * S
    info = plsc.get_sparse_core_info()
    NW = info.num_cores * info.num_subcores
    idx3 = indices.reshape(NW, (B // NW) // _C, _C).astype(jnp.int32)
    out = _gather_rows(emb_weight, idx3, B, D)
    return out.reshape(Bo, S, D)


# speedup vs baseline: 1.0700x; 1.0700x over previous
import functools

import jax
import jax.numpy as jnp
from jax import lax
from jax.experimental import pallas as pl
from jax.experimental.pallas import tpu as pltpu
from jax.experimental.pallas import tpu_sc as plsc

_C = 40
_NBUF = 2


def _gather_rows(table, idx3, B, D):
    info = plsc.get_sparse_core_info()
    NW = info.num_cores * info.num_subcores
    b_per_w = B // NW
    C = _C
    n_chunks = b_per_w // C
    assert b_per_w % C == 0 and n_chunks % _NBUF == 0

    mesh = plsc.VectorSubcoreMesh(core_axis_name="c", subcore_axis_name="s")

    @functools.partial(
        pl.kernel,
        mesh=mesh,
        out_type=jax.ShapeDtypeStruct((B, D), jnp.float32),
        scratch_types=[
            pltpu.VMEM((n_chunks, C), jnp.int32),
            *[pltpu.VMEM((C, D), jnp.float32) for _ in range(_NBUF)],
            *[pltpu.SemaphoreType.DMA for _ in range(2 * _NBUF)],
        ],
    )
    def k(idx_hbm, table_hbm, out_hbm, idx_v, *bufs_and_sems):
        rows = bufs_and_sems[:_NBUF]
        gsem = bufs_and_sems[_NBUF : 2 * _NBUF]
        osem = bufs_and_sems[2 * _NBUF :]
        wid = lax.axis_index("s") * info.num_cores + lax.axis_index("c")
        base = pl.multiple_of(wid * b_per_w, b_per_w)

        pltpu.sync_copy(idx_hbm.at[wid], idx_v)
        pltpu.async_copy(table_hbm.at[idx_v.at[0]], rows[0], gsem[0])

        def body(g, carry):
            for b in range(_NBUF):
                j = g * _NBUF + b
                pltpu.make_async_copy(
                    table_hbm.at[idx_v.at[j]], rows[b], gsem[b]
                ).wait()
                pltpu.async_copy(
                    rows[b], out_hbm.at[pl.ds(base + j * C, C)], osem[b]
                )
                jn = jnp.minimum(j + 1, n_chunks - 1)
                nb = (b + 1) % _NBUF

                @pl.when(jnp.logical_or(g > 0, b >= _NBUF - 1))
                def _wait_prev_out():
                    pltpu.make_async_copy(
                        rows[nb], out_hbm.at[pl.ds(base, C)], osem[nb]
                    ).wait()

                pltpu.async_copy(
                    table_hbm.at[idx_v.at[jn]], rows[nb], gsem[nb]
                )
            return carry

        lax.fori_loop(0, n_chunks // _NBUF, body, 0)
        pltpu.make_async_copy(
            table_hbm.at[idx_v.at[0]], rows[0], gsem[0]
        ).wait()
        for b in range(1, _NBUF):
            pltpu.make_async_copy(
                rows[b], out_hbm.at[pl.ds(base, C)], osem[b]
            ).wait()

    return k(idx3, table)


def kernel(indices, emb_weight):
    Bo, S = indices.shape
    V, D = emb_weight.shape
    B = Bo * S
    info = plsc.get_sparse_core_info()
    NW = info.num_cores * info.num_subcores
    idx3 = indices.reshape(NW, (B // NW) // _C, _C).astype(jnp.int32)
    out = _gather_rows(emb_weight, idx3, B, D)
    return out.reshape(Bo, S, D)
